# Initial kernel scaffold; baseline (speedup 1.0000x reference)
import jax
import jax.numpy as jnp
from jax.experimental import pallas as pl


def kernel(feature_in, embed_in, tree):
    raise NotImplementedError("write your pallas kernel here")


# speedup vs baseline: 129.2791x; 129.2791x over previous
import jax
import jax.numpy as jnp
from jax import lax
from jax.experimental import pallas as pl
from jax.experimental.pallas import tpu as pltpu
from jax.experimental.pallas import tpu_sc as plsc

_SIGMA = 0.02
_B = 4
_V = 4096
_C = 96
_CP = 112
_R = 12
_NC = 2
_NS = 16
_CHUNK = _V // _NS
_NIDX = _CHUNK // 128
_L = 16
_NG = _CHUNK // _L


def _row_scale_inplace(buf, scale_ref):
    def body(grp, _):
        r0 = grp * _L
        sv = scale_ref[pl.ds(r0, _L)]
        for l in range(_L):
            sc = jnp.broadcast_to(sv[l], (_L,))
            for j in range(_CP // _L):
                s_ = pl.ds(j * _L, _L)
                buf[r0 + l, s_] = buf[r0 + l, s_] * sc
        return 0

    lax.fori_loop(0, _NG, body, 0)


def _tree_filter_kernel(x_hbm, e_hbm, tree2_hbm, out_hbm,
                        s_sh, u_tab, q_tab, m_sh,
                        sl, g, ufull, qfull, uchunk, qchunk, qchunk2, mchunk,
                        sem):
    cid = lax.axis_index("c")
    tib = lax.axis_index("s")
    base = tib * _CHUNK
    rows = pl.ds(base, _CHUNK)
    iota = lax.iota(jnp.int32, _L)

    def rep_body(rep, _):
        b = cid + _NC * rep
        hrows = pl.ds(b * _V + base, _CHUNK)
        hoff = jnp.broadcast_to(b * _V, (_L,)).astype(jnp.int32)

        def tab_rows(k):
            return pl.ds((b * _R + k) * _V + base, _CHUNK)

        def tab_full(k):
            return pl.ds((b * _R + k) * _V, _V)

        plsc.subcore_barrier()

        pltpu.sync_copy(x_hbm.at[hrows], sl)
        pltpu.sync_copy(sl, s_sh.at[rows])

        pltpu.sync_copy(
            tree2_hbm.at[pl.ds((b * _V) // 128 + tib * _NIDX, _NIDX)],
            qchunk2)

        def qc_body(i, _):
            s_ = pl.ds((i % 8) * _L, _L)
            qchunk[pl.ds(i * _L, _L)] = qchunk2[i // 8, s_]
            qchunk2[i // 8, s_] = qchunk2[i // 8, s_] + hoff
            return 0

        lax.fori_loop(0, _NG, qc_body, 0)

        pltpu.sync_copy(e_hbm.at[hrows], sl)
        copies = [
            pltpu.async_copy(e_hbm.at[qchunk2.at[j]],
                             g.at[pl.ds(j * 128, 128)], sem)
            for j in range(_NIDX)
        ]
        for c in copies:
            c.wait()

        def wgrp_body(grp, _):
            r0 = grp * _L
            d2v = jnp.zeros((_L,), jnp.float32)
            for l in range(_L):
                acc = jnp.zeros((_L,), jnp.float32)
                for j in range(_CP // _L):
                    s_ = pl.ds(j * _L, _L)
                    d = sl[r0 + l, s_] - g[r0 + l, s_]
                    acc = acc + d * d
                for sh in (1, 2, 4, 8):
                    acc = acc + acc.at[iota ^ sh].get(
                        mode="promise_in_bounds")
                d2v = jnp.where(iota == l, acc, d2v)
            wv = jnp.exp(d2v * (-1.0 / _SIGMA))
            row0 = (iota == 0) & jnp.broadcast_to(base + r0 == 0, (_L,))
            uchunk[pl.ds(r0, _L)] = jnp.where(row0, 0.0, wv)
            mchunk[pl.ds(r0, _L)] = jnp.where(row0, 1.0, 1.0 - wv * wv)
            return 0

        lax.fori_loop(0, _NG, wgrp_body, 0)

        pltpu.sync_copy(uchunk, u_tab.at[tab_rows(0)])
        pltpu.sync_copy(qchunk, q_tab.at[tab_rows(0)])
        pltpu.sync_copy(mchunk, m_sh.at[rows])

        def build_body(k, _):
            plsc.subcore_barrier()
            pltpu.sync_copy(u_tab.at[tab_full(k - 1)], ufull)
            pltpu.sync_copy(q_tab.at[tab_full(k - 1)], qfull)

            def grp(i, _):
                s_ = pl.ds(i * _L, _L)
                qv = qfull[pl.ds(base + i * _L, _L)]
                uv = ufull[pl.ds(base + i * _L, _L)]
                uq = plsc.load_gather(ufull, [qv])
                qq = plsc.load_gather(qfull, [qv])
                uchunk[s_] = uv * uq
                qchunk[s_] = qq
                return 0

            lax.fori_loop(0, _NG, grp, 0)
            pltpu.sync_copy(uchunk, u_tab.at[tab_rows(k)])
            pltpu.sync_copy(qchunk, q_tab.at[tab_rows(k)])
            return 0

        lax.fori_loop(1, _R, build_body, 0)

        def up_body(kk, _):
            k = _R - 1 - kk
            plsc.subcore_barrier()
            pltpu.sync_copy(s_sh.at[rows], sl)
            pltpu.sync_copy(u_tab.at[tab_rows(k)], uchunk)
            pltpu.sync_copy(q_tab.at[tab_rows(k)], qchunk)

            def qc2(i, _):
                qchunk2[i // 8, pl.ds((i % 8) * _L, _L)] = (
                    qchunk[pl.ds(i * _L, _L)])
                return 0

            lax.fori_loop(0, _NG, qc2, 0)
            _row_scale_inplace(sl, uchunk)
            plsc.subcore_barrier()
            copies = [
                pltpu.async_copy(sl.at[pl.ds(j * 128, 128)],
                                 s_sh.at[qchunk2.at[j]], sem, add=True)
                for j in range(_NIDX)
            ]
            for c in copies:
                c.wait()
            return 0

        lax.fori_loop(0, _R, up_body, 0)

        plsc.subcore_barrier()
        pltpu.sync_copy(s_sh.at[rows], sl)
        pltpu.sync_copy(m_sh.at[rows], mchunk)
        _row_scale_inplace(sl, mchunk)
        pltpu.sync_copy(sl, s_sh.at[rows])

        def down_body(k, _):
            plsc.subcore_barrier()
            pltpu.sync_copy(u_tab.at[tab_rows(k)], uchunk)
            pltpu.sync_copy(q_tab.at[tab_rows(k)], qchunk)
            copies = [
                pltpu.async_copy(s_sh.at[qchunk.at[pl.ds(j * 128, 128)]],
                                 g.at[pl.ds(j * 128, 128)], sem)
                for j in range(_NIDX)
            ]
            pltpu.sync_copy(s_sh.at[rows], sl)
            for c in copies:
                c.wait()
            plsc.subcore_barrier()

            def fma_body(grp, _):
                r0 = grp * _L
                uv = uchunk[pl.ds(r0, _L)]
                for l in range(_L):
                    sc = jnp.broadcast_to(uv[l], (_L,))
                    for j in range(_CP // _L):
                        s_ = pl.ds(j * _L, _L)
                        sl[r0 + l, s_] = sl[r0 + l, s_] + sc * g[r0 + l, s_]
                return 0

            lax.fori_loop(0, _NG, fma_body, 0)
            pltpu.sync_copy(sl, s_sh.at[rows])
            return 0

        lax.fori_loop(0, _R, down_body, 0)

        def out_body(r, _):
            invv = 1.0 / sl[r, pl.ds(_C, _L)]
            inv = jnp.broadcast_to(invv[0], (_L,))
            for j in range(_CP // _L):
                s_ = pl.ds(j * _L, _L)
                sl[r, s_] = sl[r, s_] * inv
            return 0

        lax.fori_loop(0, _CHUNK, out_body, 0)
        pltpu.sync_copy(sl, out_hbm.at[hrows])
        return 0

    lax.fori_loop(0, _B // _NC, rep_body, 0)


_mesh = plsc.VectorSubcoreMesh(
    core_axis_name="c", subcore_axis_name="s", num_cores=_NC, num_subcores=_NS)

_filter_call = pl.kernel(
    _tree_filter_kernel,
    out_type=jax.ShapeDtypeStruct((_B * _V, _CP), jnp.float32),
    mesh=_mesh,
    compiler_params=pltpu.CompilerParams(
        needs_layout_passes=False, use_tc_tiling_on_sc=False),
    scratch_types=[
        pltpu.VMEM_SHARED((_V, _CP), jnp.float32),
        pltpu.HBM((_B * _R * _V,), jnp.float32),
        pltpu.HBM((_B * _R * _V,), jnp.int32),
        pltpu.VMEM_SHARED((_V,), jnp.float32),
        pltpu.VMEM((_CHUNK, _CP), jnp.float32),
        pltpu.VMEM((_CHUNK, _CP), jnp.float32),
        pltpu.VMEM((_V,), jnp.float32),
        pltpu.VMEM((_V,), jnp.int32),
        pltpu.VMEM((_CHUNK,), jnp.float32),
        pltpu.VMEM((_CHUNK,), jnp.int32),
        pltpu.VMEM((_NIDX, 128), jnp.int32),
        pltpu.VMEM((_CHUNK,), jnp.float32),
        pltpu.SemaphoreType.DMA,
    ],
)


@jax.jit
def kernel(feature_in, embed_in, tree):
    B, C, H, W = feature_in.shape
    V = H * W
    f = feature_in.reshape(B, C, V).transpose(0, 2, 1)
    ones = jnp.ones((B, V, 1), jnp.float32)
    zpad = jnp.zeros((B, V, _CP - C - 1), jnp.float32)
    x = jnp.concatenate([f, ones, zpad], axis=2).reshape(B * V, _CP)
    e = embed_in.reshape(B, C, V).transpose(0, 2, 1)
    e = jnp.concatenate([e, jnp.zeros((B, V, _CP - C), jnp.float32)], axis=2)
    e = e.reshape(B * V, _CP)
    tree2 = tree.astype(jnp.int32).reshape(B * V // 128, 128)
    out = _filter_call(x, e, tree2)
    out = out.reshape(B, V, _CP)
    return out[:, :, :_C].transpose(0, 2, 1).reshape(B, C, H, W)


# speedup vs baseline: 247.5064x; 1.9145x over previous
import jax
import jax.numpy as jnp
from jax import lax
from jax.experimental import pallas as pl
from jax.experimental.pallas import tpu as pltpu
from jax.experimental.pallas import tpu_sc as plsc

_SIGMA = 0.02
_B = 4
_V = 4096
_C = 96
_CP = 112
_R = 12
_NC = 2
_NS = 16
_CHUNK = _V // _NS
_NIDX = _CHUNK // 128
_L = 16
_NG = _CHUNK // _L


def _row_scale_inplace(buf, scale_ref):
    def body(grp, _):
        r0 = grp * _L
        sv = scale_ref[pl.ds(r0, _L)]
        for l in range(_L):
            sc = jnp.broadcast_to(sv[l], (_L,))
            for j in range(_CP // _L):
                s_ = pl.ds(j * _L, _L)
                buf[r0 + l, s_] = buf[r0 + l, s_] * sc
        return 0

    lax.fori_loop(0, _NG, body, 0)


def _tree_filter_kernel(x_hbm, e_hbm, tree2_hbm, out_hbm,
                        s_sh, u_tab, q_tab, m_sh,
                        sl, g, ufull, qfull, uchunk, qchunk, qchunk2, mchunk,
                        flags_sh, fbuf, sem):
    cid = lax.axis_index("c")
    tib = lax.axis_index("s")
    base = tib * _CHUNK
    rows = pl.ds(base, _CHUNK)
    iota = lax.iota(jnp.int32, _L)

    def rep_body(rep, _):
        b = cid + _NC * rep
        hrows = pl.ds(b * _V + base, _CHUNK)
        hoff = jnp.broadcast_to(b * _V, (_L,)).astype(jnp.int32)

        def tab_rows(k):
            return pl.ds((b * _R + k) * _V + base, _CHUNK)

        def tab_full(k):
            return pl.ds((b * _R + k) * _V, _V)

        plsc.subcore_barrier()

        pltpu.sync_copy(x_hbm.at[hrows], sl)
        pltpu.sync_copy(sl, s_sh.at[rows])

        pltpu.sync_copy(
            tree2_hbm.at[pl.ds((b * _V) // 128 + tib * _NIDX, _NIDX)],
            qchunk2)

        def qc_body(i, _):
            s_ = pl.ds((i % 8) * _L, _L)
            qchunk[pl.ds(i * _L, _L)] = qchunk2[i // 8, s_]
            qchunk2[i // 8, s_] = qchunk2[i // 8, s_] + hoff
            return 0

        lax.fori_loop(0, _NG, qc_body, 0)

        pltpu.sync_copy(e_hbm.at[hrows], sl)
        copies = [
            pltpu.async_copy(e_hbm.at[qchunk2.at[j]],
                             g.at[pl.ds(j * 128, 128)], sem)
            for j in range(_NIDX)
        ]
        for c in copies:
            c.wait()

        def wgrp_body(grp, maxv):
            r0 = grp * _L
            d2v = jnp.zeros((_L,), jnp.float32)
            for l in range(_L):
                acc = jnp.zeros((_L,), jnp.float32)
                for j in range(_CP // _L):
                    s_ = pl.ds(j * _L, _L)
                    d = sl[r0 + l, s_] - g[r0 + l, s_]
                    acc = acc + d * d
                for sh in (1, 2, 4, 8):
                    acc = acc + acc.at[iota ^ sh].get(
                        mode="promise_in_bounds")
                d2v = jnp.where(iota == l, acc, d2v)
            wv = jnp.exp(d2v * (-1.0 / _SIGMA))
            row0 = (iota == 0) & jnp.broadcast_to(base + r0 == 0, (_L,))
            u1 = jnp.where(row0, 0.0, wv)
            uchunk[pl.ds(r0, _L)] = u1
            mchunk[pl.ds(r0, _L)] = jnp.where(row0, 1.0, 1.0 - wv * wv)
            return jnp.maximum(maxv, u1)

        maxv = lax.fori_loop(0, _NG, wgrp_body, jnp.zeros((_L,), jnp.float32))
        fbuf[pl.ds(0, _L)] = maxv
        pltpu.sync_copy(fbuf, flags_sh.at[pl.ds(tib * _L, _L)])

        pltpu.sync_copy(uchunk, u_tab.at[tab_rows(0)])
        pltpu.sync_copy(qchunk, q_tab.at[tab_rows(0)])
        pltpu.sync_copy(mchunk, m_sh.at[rows])

        def build_body(k, _):
            plsc.subcore_barrier()
            pltpu.sync_copy(u_tab.at[tab_full(k - 1)], ufull)
            pltpu.sync_copy(q_tab.at[tab_full(k - 1)], qfull)

            def grp(i, maxv):
                s_ = pl.ds(i * _L, _L)
                qv = qfull[pl.ds(base + i * _L, _L)]
                uv = ufull[pl.ds(base + i * _L, _L)]
                uq = plsc.load_gather(ufull, [qv])
                qq = plsc.load_gather(qfull, [qv])
                u2 = uv * uq
                uchunk[s_] = u2
                qchunk[s_] = qq
                return jnp.maximum(maxv, u2)

            maxv = lax.fori_loop(0, _NG, grp, jnp.zeros((_L,), jnp.float32))
            fbuf[pl.ds(0, _L)] = maxv
            pltpu.sync_copy(fbuf,
                            flags_sh.at[pl.ds(k * _NS * _L + tib * _L, _L)])
            pltpu.sync_copy(uchunk, u_tab.at[tab_rows(k)])
            pltpu.sync_copy(qchunk, q_tab.at[tab_rows(k)])
            return 0

        lax.fori_loop(1, _R, build_body, 0)

        plsc.subcore_barrier()
        pltpu.sync_copy(flags_sh, ufull.at[pl.ds(0, _R * _NS * _L)])
        nv = jnp.zeros((_L,), jnp.int32)
        for k in range(_R):
            mk = ufull[pl.ds(k * _NS * _L, _L)]
            for t in range(1, _NS):
                mk = jnp.maximum(mk, ufull[pl.ds(k * _NS * _L + t * _L, _L)])
            for sh in (1, 2, 4, 8):
                mk = jnp.maximum(
                    mk, mk.at[iota ^ sh].get(mode="promise_in_bounds"))
            nv = nv + jnp.where(mk > 0.0, 1, 0)
        n_eff = nv[0]

        def up_body(kk, _):
            k = n_eff - 1 - kk
            plsc.subcore_barrier()
            pltpu.sync_copy(s_sh.at[rows], sl)
            pltpu.sync_copy(u_tab.at[tab_rows(k)], uchunk)
            pltpu.sync_copy(q_tab.at[tab_rows(k)], qchunk)

            def qc2(i, _):
                qchunk2[i // 8, pl.ds((i % 8) * _L, _L)] = (
                    qchunk[pl.ds(i * _L, _L)])
                return 0

            lax.fori_loop(0, _NG, qc2, 0)
            _row_scale_inplace(sl, uchunk)
            plsc.subcore_barrier()
            copies = [
                pltpu.async_copy(sl.at[pl.ds(j * 128, 128)],
                                 s_sh.at[qchunk2.at[j]], sem, add=True)
                for j in range(_NIDX)
            ]
            for c in copies:
                c.wait()
            return 0

        lax.fori_loop(0, n_eff, up_body, 0)

        plsc.subcore_barrier()
        pltpu.sync_copy(s_sh.at[rows], sl)
        pltpu.sync_copy(m_sh.at[rows], mchunk)
        _row_scale_inplace(sl, mchunk)
        pltpu.sync_copy(sl, s_sh.at[rows])

        def down_body(k, _):
            plsc.subcore_barrier()
            pltpu.sync_copy(u_tab.at[tab_rows(k)], uchunk)
            pltpu.sync_copy(q_tab.at[tab_rows(k)], qchunk)
            copies = [
                pltpu.async_copy(s_sh.at[qchunk.at[pl.ds(j * 128, 128)]],
                                 g.at[pl.ds(j * 128, 128)], sem)
                for j in range(_NIDX)
            ]
            pltpu.sync_copy(s_sh.at[rows], sl)
            for c in copies:
                c.wait()
            plsc.subcore_barrier()

            def fma_body(grp, _):
                r0 = grp * _L
                uv = uchunk[pl.ds(r0, _L)]
                for l in range(_L):
                    sc = jnp.broadcast_to(uv[l], (_L,))
                    for j in range(_CP // _L):
                        s_ = pl.ds(j * _L, _L)
                        sl[r0 + l, s_] = sl[r0 + l, s_] + sc * g[r0 + l, s_]
                return 0

            lax.fori_loop(0, _NG, fma_body, 0)
            pltpu.sync_copy(sl, s_sh.at[rows])
            return 0

        lax.fori_loop(0, n_eff, down_body, 0)

        def out_body(r, _):
            invv = 1.0 / sl[r, pl.ds(_C, _L)]
            inv = jnp.broadcast_to(invv[0], (_L,))
            for j in range(_CP // _L):
                s_ = pl.ds(j * _L, _L)
                sl[r, s_] = sl[r, s_] * inv
            return 0

        lax.fori_loop(0, _CHUNK, out_body, 0)
        pltpu.sync_copy(sl, out_hbm.at[hrows])
        return 0

    lax.fori_loop(0, _B // _NC, rep_body, 0)


_mesh = plsc.VectorSubcoreMesh(
    core_axis_name="c", subcore_axis_name="s", num_cores=_NC, num_subcores=_NS)

_filter_call = pl.kernel(
    _tree_filter_kernel,
    out_type=jax.ShapeDtypeStruct((_B * _V, _CP), jnp.float32),
    mesh=_mesh,
    compiler_params=pltpu.CompilerParams(
        needs_layout_passes=False, use_tc_tiling_on_sc=False),
    scratch_types=[
        pltpu.VMEM_SHARED((_V, _CP), jnp.float32),
        pltpu.HBM((_B * _R * _V,), jnp.float32),
        pltpu.HBM((_B * _R * _V,), jnp.int32),
        pltpu.VMEM_SHARED((_V,), jnp.float32),
        pltpu.VMEM((_CHUNK, _CP), jnp.float32),
        pltpu.VMEM((_CHUNK, _CP), jnp.float32),
        pltpu.VMEM((_V,), jnp.float32),
        pltpu.VMEM((_V,), jnp.int32),
        pltpu.VMEM((_CHUNK,), jnp.float32),
        pltpu.VMEM((_CHUNK,), jnp.int32),
        pltpu.VMEM((_NIDX, 128), jnp.int32),
        pltpu.VMEM((_CHUNK,), jnp.float32),
        pltpu.VMEM_SHARED((_R * _NS * _L,), jnp.float32),
        pltpu.VMEM((_L,), jnp.float32),
        pltpu.SemaphoreType.DMA,
    ],
)


@jax.jit
def kernel(feature_in, embed_in, tree):
    B, C, H, W = feature_in.shape
    V = H * W
    f = feature_in.reshape(B, C, V).transpose(0, 2, 1)
    ones = jnp.ones((B, V, 1), jnp.float32)
    zpad = jnp.zeros((B, V, _CP - C - 1), jnp.float32)
    x = jnp.concatenate([f, ones, zpad], axis=2).reshape(B * V, _CP)
    e = embed_in.reshape(B, C, V).transpose(0, 2, 1)
    e = jnp.concatenate([e, jnp.zeros((B, V, _CP - C), jnp.float32)], axis=2)
    e = e.reshape(B * V, _CP)
    tree2 = tree.astype(jnp.int32).reshape(B * V // 128, 128)
    out = _filter_call(x, e, tree2)
    out = out.reshape(B, V, _CP)
    return out[:, :, :_C].transpose(0, 2, 1).reshape(B, C, H, W)


# speedup vs baseline: 292.4813x; 1.1817x over previous
import jax
import jax.numpy as jnp
from jax import lax
from jax.experimental import pallas as pl
from jax.experimental.pallas import tpu as pltpu
from jax.experimental.pallas import tpu_sc as plsc

_SIGMA = 0.02
_B = 4
_V = 4096
_C = 96
_CP = 112
_R = 12
_NC = 2
_NS = 16
_CHUNK = _V // _NS
_NIDX = _CHUNK // 128
_L = 16
_NG = _CHUNK // _L


def _row_scale_inplace(buf, scale_ref):
    def body(grp, _):
        r0 = grp * _L
        sv = scale_ref[pl.ds(r0, _L)]
        for l in range(_L):
            sc = jnp.broadcast_to(sv[l], (_L,))
            for j in range(_CP // _L):
                s_ = pl.ds(j * _L, _L)
                buf[r0 + l, s_] = buf[r0 + l, s_] * sc
        return 0

    lax.fori_loop(0, _NG, body, 0)


def _tree_filter_kernel(x_hbm, e_hbm, tree2_hbm, out_hbm,
                        s_sh, u_tab, q_tab, m_sh, flags_sh,
                        sl, g, ufull, qfull,
                        uchunk, qchunk, qchunk2, mchunk, fbuf, sem):
    cid = lax.axis_index("c")
    tib = lax.axis_index("s")
    base = tib * _CHUNK
    rows = pl.ds(base, _CHUNK)
    iota = lax.iota(jnp.int32, _L)

    def rep_body(rep, _):
        b = cid + _NC * rep
        hrows = pl.ds(b * _V + base, _CHUNK)
        hoff = jnp.broadcast_to(b * _V, (_L,)).astype(jnp.int32)

        def tab_rows(k):
            return pl.ds((b * _R + k) * _V + base, _CHUNK)

        def tab_full(k):
            return pl.ds((b * _R + k) * _V, _V)

        plsc.subcore_barrier()

        pltpu.sync_copy(x_hbm.at[hrows], sl)
        pltpu.sync_copy(sl, s_sh.at[rows])

        pltpu.sync_copy(
            tree2_hbm.at[pl.ds((b * _V) // 128 + tib * _NIDX, _NIDX)],
            qchunk2)

        def qc_body(i, _):
            s_ = pl.ds((i % 8) * _L, _L)
            qchunk[pl.ds(i * _L, _L)] = qchunk2[i // 8, s_]
            qchunk2[i // 8, s_] = qchunk2[i // 8, s_] + hoff
            return 0

        lax.fori_loop(0, _NG, qc_body, 0)

        pltpu.sync_copy(e_hbm.at[hrows], sl)
        copies = [
            pltpu.async_copy(e_hbm.at[qchunk2.at[j]],
                             g.at[pl.ds(j * 128, 128)], sem)
            for j in range(_NIDX)
        ]
        for c in copies:
            c.wait()

        def wgrp_body(grp, _):
            r0 = grp * _L
            d2v = jnp.zeros((_L,), jnp.float32)
            for l in range(_L):
                acc = jnp.zeros((_L,), jnp.float32)
                for j in range(_CP // _L):
                    s_ = pl.ds(j * _L, _L)
                    d = sl[r0 + l, s_] - g[r0 + l, s_]
                    acc = acc + d * d
                for sh in (1, 2, 4, 8):
                    acc = acc + acc.at[iota ^ sh].get(
                        mode="promise_in_bounds")
                d2v = jnp.where(iota == l, acc, d2v)
            wv = jnp.exp(d2v * (-1.0 / _SIGMA))
            row0 = (iota == 0) & jnp.broadcast_to(base + r0 == 0, (_L,))
            uchunk[pl.ds(r0, _L)] = jnp.where(row0, 0.0, wv)
            mchunk[pl.ds(r0, _L)] = jnp.where(row0, 1.0, 1.0 - wv * wv)
            return 0

        lax.fori_loop(0, _NG, wgrp_body, 0)

        pltpu.sync_copy(uchunk, u_tab.at[tab_rows(0)])
        pltpu.sync_copy(qchunk, q_tab.at[tab_rows(0)])
        pltpu.sync_copy(mchunk, m_sh.at[rows])

        def build_cond(carry):
            return carry[1]

        def build_body(carry):
            k, _, nz = carry
            plsc.subcore_barrier()
            pltpu.sync_copy(u_tab.at[tab_full(k - 1)], ufull)
            pltpu.sync_copy(q_tab.at[tab_full(k - 1)], qfull)

            def grp(i, maxv):
                s_ = pl.ds(i * _L, _L)
                qv = qfull[pl.ds(base + i * _L, _L)]
                uv = ufull[pl.ds(base + i * _L, _L)]
                uq = plsc.load_gather(ufull, [qv])
                qq = plsc.load_gather(qfull, [qv])
                u2 = uv * uq
                uchunk[s_] = u2
                qchunk[s_] = qq
                return jnp.maximum(maxv, u2)

            maxv = lax.fori_loop(0, _NG, grp,
                                 jnp.zeros((_L,), jnp.float32))
            fbuf[pl.ds(0, _L)] = maxv
            pltpu.sync_copy(fbuf, flags_sh.at[pl.ds(tib * _L, _L)])
            pltpu.sync_copy(uchunk, u_tab.at[tab_rows(k)])
            pltpu.sync_copy(qchunk, q_tab.at[tab_rows(k)])
            plsc.subcore_barrier()
            pltpu.sync_copy(flags_sh, ufull.at[pl.ds(0, _NS * _L)])
            mk = ufull[pl.ds(0, _L)]
            for t in range(1, _NS):
                mk = jnp.maximum(mk, ufull[pl.ds(t * _L, _L)])
            for sh in (1, 2, 4, 8):
                mk = jnp.maximum(
                    mk, mk.at[iota ^ sh].get(mode="promise_in_bounds"))
            nonzero = mk[0] > 0.0
            cont = jnp.logical_and(nonzero, k + 1 < _R)
            return (k + 1, cont, nz + jnp.where(nonzero, 1, 0))

        _, _, n_eff = lax.while_loop(
            build_cond, build_body,
            (jnp.int32(1), jnp.bool_(True), jnp.int32(1)))

        def up_body(kk, _):
            k = n_eff - 1 - kk
            plsc.subcore_barrier()
            pltpu.sync_copy(s_sh.at[rows], sl)
            pltpu.sync_copy(u_tab.at[tab_rows(k)], uchunk)
            pltpu.sync_copy(q_tab.at[tab_rows(k)], qchunk)

            def qc2(i, _):
                qchunk2[i // 8, pl.ds((i % 8) * _L, _L)] = (
                    qchunk[pl.ds(i * _L, _L)])
                return 0

            lax.fori_loop(0, _NG, qc2, 0)
            _row_scale_inplace(sl, uchunk)
            plsc.subcore_barrier()
            copies = [
                pltpu.async_copy(sl.at[pl.ds(j * 128, 128)],
                                 s_sh.at[qchunk2.at[j]], sem, add=True)
                for j in range(_NIDX)
            ]
            for c in copies:
                c.wait()
            return 0

        lax.fori_loop(0, n_eff, up_body, 0)

        plsc.subcore_barrier()
        pltpu.sync_copy(s_sh.at[rows], sl)
        pltpu.sync_copy(m_sh.at[rows], mchunk)
        _row_scale_inplace(sl, mchunk)
        pltpu.sync_copy(sl, s_sh.at[rows])

        def down_body(k, _):
            plsc.subcore_barrier()
            pltpu.sync_copy(u_tab.at[tab_rows(k)], uchunk)
            pltpu.sync_copy(q_tab.at[tab_rows(k)], qchunk)
            copies = [
                pltpu.async_copy(s_sh.at[qchunk.at[pl.ds(j * 128, 128)]],
                                 g.at[pl.ds(j * 128, 128)], sem)
                for j in range(_NIDX)
            ]
            for c in copies:
                c.wait()
            plsc.subcore_barrier()

            def fma_body(grp, _):
                r0 = grp * _L
                uv = uchunk[pl.ds(r0, _L)]
                for l in range(_L):
                    sc = jnp.broadcast_to(uv[l], (_L,))
                    for j in range(_CP // _L):
                        s_ = pl.ds(j * _L, _L)
                        sl[r0 + l, s_] = sl[r0 + l, s_] + sc * g[r0 + l, s_]
                return 0

            lax.fori_loop(0, _NG, fma_body, 0)
            pltpu.sync_copy(sl, s_sh.at[rows])
            return 0

        lax.fori_loop(0, n_eff, down_body, 0)

        def out_body(r, _):
            invv = 1.0 / sl[r, pl.ds(_C, _L)]
            inv = jnp.broadcast_to(invv[0], (_L,))
            for j in range(_CP // _L):
                s_ = pl.ds(j * _L, _L)
                sl[r, s_] = sl[r, s_] * inv
            return 0

        lax.fori_loop(0, _CHUNK, out_body, 0)
        pltpu.sync_copy(sl, out_hbm.at[hrows])
        return 0

    lax.fori_loop(0, _B // _NC, rep_body, 0)


_mesh = plsc.VectorSubcoreMesh(
    core_axis_name="c", subcore_axis_name="s", num_cores=_NC, num_subcores=_NS)

_filter_call = pl.kernel(
    _tree_filter_kernel,
    out_type=jax.ShapeDtypeStruct((_B * _V, _CP), jnp.float32),
    mesh=_mesh,
    compiler_params=pltpu.CompilerParams(
        needs_layout_passes=False, use_tc_tiling_on_sc=False),
    scratch_types=[
        pltpu.VMEM_SHARED((_V, _CP), jnp.float32),
        pltpu.HBM((_B * _R * _V,), jnp.float32),
        pltpu.HBM((_B * _R * _V,), jnp.int32),
        pltpu.VMEM_SHARED((_V,), jnp.float32),
        pltpu.VMEM_SHARED((_NS * _L,), jnp.float32),
        pltpu.VMEM((_CHUNK, _CP), jnp.float32),
        pltpu.VMEM((_CHUNK, _CP), jnp.float32),
        pltpu.VMEM((_V,), jnp.float32),
        pltpu.VMEM((_V,), jnp.int32),
        pltpu.VMEM((_CHUNK,), jnp.float32),
        pltpu.VMEM((_CHUNK,), jnp.int32),
        pltpu.VMEM((_NIDX, 128), jnp.int32),
        pltpu.VMEM((_CHUNK,), jnp.float32),
        pltpu.VMEM((_L,), jnp.float32),
        pltpu.SemaphoreType.DMA,
    ],
)


@jax.jit
def kernel(feature_in, embed_in, tree):
    B, C, H, W = feature_in.shape
    V = H * W
    f = feature_in.reshape(B, C, V).transpose(0, 2, 1)
    ones = jnp.ones((B, V, 1), jnp.float32)
    zpad = jnp.zeros((B, V, _CP - C - 1), jnp.float32)
    x = jnp.concatenate([f, ones, zpad], axis=2).reshape(B * V, _CP)
    e = embed_in.reshape(B, C, V).transpose(0, 2, 1)
    e = jnp.concatenate([e, jnp.zeros((B, V, _CP - C), jnp.float32)], axis=2)
    e = e.reshape(B * V, _CP)
    tree2 = tree.astype(jnp.int32).reshape(B * V // 128, 128)
    out = _filter_call(x, e, tree2)
    out = out.reshape(B, V, _CP)
    return out[:, :, :_C].transpose(0, 2, 1).reshape(B, C, H, W)


# speedup vs baseline: 292.9285x; 1.0015x over previous
import jax
import jax.numpy as jnp
from jax import lax
from jax.experimental import pallas as pl
from jax.experimental.pallas import tpu as pltpu
from jax.experimental.pallas import tpu_sc as plsc

_SIGMA = 0.02
_B = 4
_V = 4096
_C = 96
_CP = 112
_R = 12
_NC = 2
_NS = 16
_CHUNK = _V // _NS
_NIDX = _CHUNK // 128
_L = 16
_NG = _CHUNK // _L


def _row_scale_inplace(buf, scale_ref):
    def body(grp, _):
        r0 = grp * _L
        sv = scale_ref[pl.ds(r0, _L)]
        for l in range(_L):
            sc = jnp.broadcast_to(sv[l], (_L,))
            for j in range(_CP // _L):
                s_ = pl.ds(j * _L, _L)
                buf[r0 + l, s_] = buf[r0 + l, s_] * sc
        return 0

    lax.fori_loop(0, _NG, body, 0)


def _tree_filter_kernel(x_hbm, e_hbm, tree2_hbm, out_hbm,
                        s_sh, u_tab, q_tab, m_sh, flags_sh,
                        sl, g, xo, ufull, qfull,
                        uchunk, qchunk, qchunk2, mchunk, fbuf, sem):
    cid = lax.axis_index("c")
    tib = lax.axis_index("s")
    base = tib * _CHUNK
    rows = pl.ds(base, _CHUNK)
    iota = lax.iota(jnp.int32, _L)

    def rep_body(rep, _):
        b = cid + _NC * rep
        hrows = pl.ds(b * _V + base, _CHUNK)
        hoff = jnp.broadcast_to(b * _V, (_L,)).astype(jnp.int32)

        def tab_rows(k):
            return pl.ds((b * _R + k) * _V + base, _CHUNK)

        def tab_full(k):
            return pl.ds((b * _R + k) * _V, _V)

        plsc.subcore_barrier()

        pltpu.sync_copy(x_hbm.at[hrows], xo)
        onesv = jnp.where(iota == 0, 1.0, 0.0).astype(jnp.float32)

        def xg_body(grp, _):
            r0 = grp * _L
            for l in range(_L):
                for j in range(_C // _L):
                    s_ = pl.ds(j * _L, _L)
                    sl[r0 + l, s_] = xo[r0 + l, s_]
                sl[r0 + l, pl.ds(_C, _L)] = onesv
            return 0

        lax.fori_loop(0, _NG, xg_body, 0)
        pltpu.sync_copy(sl, s_sh.at[rows])

        pltpu.sync_copy(
            tree2_hbm.at[pl.ds((b * _V) // 128 + tib * _NIDX, _NIDX)],
            qchunk2)

        def qc_body(i, _):
            s_ = pl.ds((i % 8) * _L, _L)
            qchunk[pl.ds(i * _L, _L)] = qchunk2[i // 8, s_]
            qchunk2[i // 8, s_] = qchunk2[i // 8, s_] + hoff
            return 0

        lax.fori_loop(0, _NG, qc_body, 0)

        pltpu.sync_copy(e_hbm.at[hrows], sl)
        copies = [
            pltpu.async_copy(e_hbm.at[qchunk2.at[j]],
                             g.at[pl.ds(j * 128, 128)], sem)
            for j in range(_NIDX)
        ]
        for c in copies:
            c.wait()

        def wgrp_body(grp, _):
            r0 = grp * _L
            d2v = jnp.zeros((_L,), jnp.float32)
            for l in range(_L):
                acc = jnp.zeros((_L,), jnp.float32)
                for j in range(_CP // _L):
                    s_ = pl.ds(j * _L, _L)
                    d = sl[r0 + l, s_] - g[r0 + l, s_]
                    acc = acc + d * d
                for sh in (1, 2, 4, 8):
                    acc = acc + acc.at[iota ^ sh].get(
                        mode="promise_in_bounds")
                d2v = jnp.where(iota == l, acc, d2v)
            wv = jnp.exp(d2v * (-1.0 / _SIGMA))
            row0 = (iota == 0) & jnp.broadcast_to(base + r0 == 0, (_L,))
            uchunk[pl.ds(r0, _L)] = jnp.where(row0, 0.0, wv)
            mchunk[pl.ds(r0, _L)] = jnp.where(row0, 1.0, 1.0 - wv * wv)
            return 0

        lax.fori_loop(0, _NG, wgrp_body, 0)

        pltpu.sync_copy(uchunk, u_tab.at[tab_rows(0)])
        pltpu.sync_copy(qchunk, q_tab.at[tab_rows(0)])
        pltpu.sync_copy(mchunk, m_sh.at[rows])

        def build_cond(carry):
            return carry[1]

        def build_body(carry):
            k, _, nz = carry
            plsc.subcore_barrier()
            pltpu.sync_copy(u_tab.at[tab_full(k - 1)], ufull)
            pltpu.sync_copy(q_tab.at[tab_full(k - 1)], qfull)

            def grp(i, maxv):
                s_ = pl.ds(i * _L, _L)
                qv = qfull[pl.ds(base + i * _L, _L)]
                uv = ufull[pl.ds(base + i * _L, _L)]
                uq = plsc.load_gather(ufull, [qv])
                qq = plsc.load_gather(qfull, [qv])
                u2 = uv * uq
                uchunk[s_] = u2
                qchunk[s_] = qq
                return jnp.maximum(maxv, u2)

            maxv = lax.fori_loop(0, _NG, grp,
                                 jnp.zeros((_L,), jnp.float32))
            fbuf[pl.ds(0, _L)] = maxv
            pltpu.sync_copy(fbuf, flags_sh.at[pl.ds(tib * _L, _L)])
            pltpu.sync_copy(uchunk, u_tab.at[tab_rows(k)])
            pltpu.sync_copy(qchunk, q_tab.at[tab_rows(k)])
            plsc.subcore_barrier()
            pltpu.sync_copy(flags_sh, ufull.at[pl.ds(0, _NS * _L)])
            mk = ufull[pl.ds(0, _L)]
            for t in range(1, _NS):
                mk = jnp.maximum(mk, ufull[pl.ds(t * _L, _L)])
            for sh in (1, 2, 4, 8):
                mk = jnp.maximum(
                    mk, mk.at[iota ^ sh].get(mode="promise_in_bounds"))
            nonzero = mk[0] > 0.0
            cont = jnp.logical_and(nonzero, k + 1 < _R)
            return (k + 1, cont, nz + jnp.where(nonzero, 1, 0))

        _, _, n_eff = lax.while_loop(
            build_cond, build_body,
            (jnp.int32(1), jnp.bool_(True), jnp.int32(1)))

        def up_body(kk, _):
            k = n_eff - 1 - kk
            plsc.subcore_barrier()
            pltpu.sync_copy(s_sh.at[rows], sl)
            pltpu.sync_copy(u_tab.at[tab_rows(k)], uchunk)
            pltpu.sync_copy(q_tab.at[tab_rows(k)], qchunk)

            def qc2(i, _):
                qchunk2[i // 8, pl.ds((i % 8) * _L, _L)] = (
                    qchunk[pl.ds(i * _L, _L)])
                return 0

            lax.fori_loop(0, _NG, qc2, 0)
            _row_scale_inplace(sl, uchunk)
            plsc.subcore_barrier()
            copies = [
                pltpu.async_copy(sl.at[pl.ds(j * 128, 128)],
                                 s_sh.at[qchunk2.at[j]], sem, add=True)
                for j in range(_NIDX)
            ]
            for c in copies:
                c.wait()
            return 0

        lax.fori_loop(0, n_eff, up_body, 0)

        plsc.subcore_barrier()
        pltpu.sync_copy(s_sh.at[rows], sl)
        pltpu.sync_copy(m_sh.at[rows], mchunk)
        _row_scale_inplace(sl, mchunk)
        pltpu.sync_copy(sl, s_sh.at[rows])

        def down_body(k, _):
            plsc.subcore_barrier()
            pltpu.sync_copy(u_tab.at[tab_rows(k)], uchunk)
            pltpu.sync_copy(q_tab.at[tab_rows(k)], qchunk)
            copies = [
                pltpu.async_copy(s_sh.at[qchunk.at[pl.ds(j * 128, 128)]],
                                 g.at[pl.ds(j * 128, 128)], sem)
                for j in range(_NIDX)
            ]
            for c in copies:
                c.wait()
            plsc.subcore_barrier()

            def fma_body(grp, _):
                r0 = grp * _L
                uv = uchunk[pl.ds(r0, _L)]
                for l in range(_L):
                    sc = jnp.broadcast_to(uv[l], (_L,))
                    for j in range(_CP // _L):
                        s_ = pl.ds(j * _L, _L)
                        sl[r0 + l, s_] = sl[r0 + l, s_] + sc * g[r0 + l, s_]
                return 0

            lax.fori_loop(0, _NG, fma_body, 0)
            pltpu.sync_copy(sl, s_sh.at[rows])
            return 0

        lax.fori_loop(0, n_eff, down_body, 0)

        def out_body(r, _):
            invv = 1.0 / sl[r, pl.ds(_C, _L)]
            inv = jnp.broadcast_to(invv[0], (_L,))
            for j in range(_CP // _L):
                s_ = pl.ds(j * _L, _L)
                sl[r, s_] = sl[r, s_] * inv
            return 0

        lax.fori_loop(0, _CHUNK, out_body, 0)
        pltpu.sync_copy(sl, out_hbm.at[hrows])
        return 0

    lax.fori_loop(0, _B // _NC, rep_body, 0)


_mesh = plsc.VectorSubcoreMesh(
    core_axis_name="c", subcore_axis_name="s", num_cores=_NC, num_subcores=_NS)

_filter_call = pl.kernel(
    _tree_filter_kernel,
    out_type=jax.ShapeDtypeStruct((_B * _V, _CP), jnp.float32),
    mesh=_mesh,
    compiler_params=pltpu.CompilerParams(
        needs_layout_passes=False, use_tc_tiling_on_sc=False),
    scratch_types=[
        pltpu.VMEM_SHARED((_V, _CP), jnp.float32),
        pltpu.HBM((_B * _R * _V,), jnp.float32),
        pltpu.HBM((_B * _R * _V,), jnp.int32),
        pltpu.VMEM_SHARED((_V,), jnp.float32),
        pltpu.VMEM_SHARED((_NS * _L,), jnp.float32),
        pltpu.VMEM((_CHUNK, _CP), jnp.float32),
        pltpu.VMEM((_CHUNK, _CP), jnp.float32),
        pltpu.VMEM((_CHUNK, _C), jnp.float32),
        pltpu.VMEM((_V,), jnp.float32),
        pltpu.VMEM((_V,), jnp.int32),
        pltpu.VMEM((_CHUNK,), jnp.float32),
        pltpu.VMEM((_CHUNK,), jnp.int32),
        pltpu.VMEM((_NIDX, 128), jnp.int32),
        pltpu.VMEM((_CHUNK,), jnp.float32),
        pltpu.VMEM((_L,), jnp.float32),
        pltpu.SemaphoreType.DMA,
    ],
)


@jax.jit
def kernel(feature_in, embed_in, tree):
    B, C, H, W = feature_in.shape
    V = H * W
    x = feature_in.reshape(B, C, V).transpose(0, 2, 1).reshape(B * V, C)
    e = embed_in.reshape(B, C, V).transpose(0, 2, 1)
    e = jnp.concatenate([e, jnp.zeros((B, V, _CP - C), jnp.float32)], axis=2)
    e = e.reshape(B * V, _CP)
    tree2 = tree.astype(jnp.int32).reshape(B * V // 128, 128)
    out = _filter_call(x, e, tree2)
    out = out.reshape(B, V, _CP)
    return out[:, :, :_C].transpose(0, 2, 1).reshape(B, C, H, W)


# speedup vs baseline: 315.5934x; 1.0774x over previous
import jax
import jax.numpy as jnp
from jax import lax
from jax.experimental import pallas as pl
from jax.experimental.pallas import tpu as pltpu
from jax.experimental.pallas import tpu_sc as plsc

_SIGMA = 0.02
_B = 4
_V = 4096
_C = 96
_CP = 112
_R = 12
_NC = 2
_NS = 16
_CHUNK = _V // _NS
_NIDX = _CHUNK // 128
_L = 16
_NG = _CHUNK // _L


def _row_scale_inplace(buf, scale_ref):
    def body(grp, _):
        r0 = grp * _L
        sv = scale_ref[pl.ds(r0, _L)]
        for l in range(_L):
            sc = jnp.broadcast_to(sv[l], (_L,))
            for j in range(_CP // _L):
                s_ = pl.ds(j * _L, _L)
                buf[r0 + l, s_] = buf[r0 + l, s_] * sc
        return 0

    lax.fori_loop(0, _NG, body, 0)


def _tree_filter_kernel(x_hbm, e_hbm, tree2_hbm, out_hbm,
                        s_sh, u_tab, q_tab, m_sh, flags_sh,
                        sl, g, xo, ufull, qfull,
                        uchunk, qchunk, qchunk2, mchunk, fbuf, sem, sem2, sem3):
    cid = lax.axis_index("c")
    tib = lax.axis_index("s")
    base = tib * _CHUNK
    rows = pl.ds(base, _CHUNK)
    iota = lax.iota(jnp.int32, _L)

    def rep_body(rep, _):
        b = cid + _NC * rep
        hrows = pl.ds(b * _V + base, _CHUNK)
        hoff = jnp.broadcast_to(b * _V, (_L,)).astype(jnp.int32)

        def tab_rows(k):
            return pl.ds((b * _R + k) * _V + base, _CHUNK)

        def tab_full(k):
            return pl.ds((b * _R + k) * _V, _V)

        plsc.subcore_barrier()

        pltpu.sync_copy(x_hbm.at[hrows], xo)
        onesv = jnp.where(iota == 0, 1.0, 0.0).astype(jnp.float32)

        def xg_body(grp, _):
            r0 = grp * _L
            for l in range(_L):
                for j in range(_C // _L):
                    s_ = pl.ds(j * _L, _L)
                    sl[r0 + l, s_] = xo[r0 + l, s_]
                sl[r0 + l, pl.ds(_C, _L)] = onesv
            return 0

        lax.fori_loop(0, _NG, xg_body, 0)
        pltpu.sync_copy(sl, s_sh.at[rows])

        pltpu.sync_copy(
            tree2_hbm.at[pl.ds((b * _V) // 128 + tib * _NIDX, _NIDX)],
            qchunk2)

        def qc_body(i, _):
            s_ = pl.ds((i % 8) * _L, _L)
            qchunk[pl.ds(i * _L, _L)] = qchunk2[i // 8, s_]
            qchunk2[i // 8, s_] = qchunk2[i // 8, s_] + hoff
            return 0

        lax.fori_loop(0, _NG, qc_body, 0)

        pltpu.sync_copy(e_hbm.at[hrows], sl)
        copies = [
            pltpu.async_copy(e_hbm.at[qchunk2.at[j]],
                             g.at[pl.ds(j * 128, 128)], sem)
            for j in range(_NIDX)
        ]
        for c in copies:
            c.wait()

        def wgrp_body(grp, _):
            r0 = grp * _L
            d2v = jnp.zeros((_L,), jnp.float32)
            for l in range(_L):
                acc = jnp.zeros((_L,), jnp.float32)
                for j in range(_CP // _L):
                    s_ = pl.ds(j * _L, _L)
                    d = sl[r0 + l, s_] - g[r0 + l, s_]
                    acc = acc + d * d
                for sh in (1, 2, 4, 8):
                    acc = acc + acc.at[iota ^ sh].get(
                        mode="promise_in_bounds")
                d2v = jnp.where(iota == l, acc, d2v)
            wv = jnp.exp(d2v * (-1.0 / _SIGMA))
            row0 = (iota == 0) & jnp.broadcast_to(base + r0 == 0, (_L,))
            uchunk[pl.ds(r0, _L)] = jnp.where(row0, 0.0, wv)
            mchunk[pl.ds(r0, _L)] = jnp.where(row0, 1.0, 1.0 - wv * wv)
            return 0

        lax.fori_loop(0, _NG, wgrp_body, 0)

        pltpu.sync_copy(uchunk, u_tab.at[tab_rows(0)])
        pltpu.sync_copy(qchunk, q_tab.at[tab_rows(0)])
        pltpu.sync_copy(mchunk, m_sh.at[rows])

        def build_cond(carry):
            return carry[1]

        def build_body(carry):
            k, _, nz = carry
            plsc.subcore_barrier()
            dls = [
                pltpu.async_copy(u_tab.at[tab_full(k - 1)], ufull, sem2),
                pltpu.async_copy(q_tab.at[tab_full(k - 1)], qfull, sem3),
            ]
            for c in dls:
                c.wait()

            def grp(i, maxv):
                s_ = pl.ds(i * _L, _L)
                qv = qfull[pl.ds(base + i * _L, _L)]
                uv = ufull[pl.ds(base + i * _L, _L)]
                uq = plsc.load_gather(ufull, [qv])
                qq = plsc.load_gather(qfull, [qv])
                u2 = uv * uq
                uchunk[s_] = u2
                qchunk[s_] = qq
                return jnp.maximum(maxv, u2)

            maxv = lax.fori_loop(0, _NG, grp,
                                 jnp.zeros((_L,), jnp.float32))
            fbuf[pl.ds(0, _L)] = maxv
            pltpu.sync_copy(fbuf, flags_sh.at[pl.ds(tib * _L, _L)])
            pltpu.sync_copy(uchunk, u_tab.at[tab_rows(k)])
            pltpu.sync_copy(qchunk, q_tab.at[tab_rows(k)])
            plsc.subcore_barrier()
            pltpu.sync_copy(flags_sh, ufull.at[pl.ds(0, _NS * _L)])
            mk = ufull[pl.ds(0, _L)]
            for t in range(1, _NS):
                mk = jnp.maximum(mk, ufull[pl.ds(t * _L, _L)])
            for sh in (1, 2, 4, 8):
                mk = jnp.maximum(
                    mk, mk.at[iota ^ sh].get(mode="promise_in_bounds"))
            nonzero = mk[0] > 0.0
            cont = jnp.logical_and(nonzero, k + 1 < _R)
            return (k + 1, cont, nz + jnp.where(nonzero, 1, 0))

        _, _, n_eff = lax.while_loop(
            build_cond, build_body,
            (jnp.int32(1), jnp.bool_(True), jnp.int32(1)))

        def up_body(kk, _):
            k = n_eff - 1 - kk
            plsc.subcore_barrier()
            dls = [
                pltpu.async_copy(s_sh.at[rows], sl, sem),
                pltpu.async_copy(u_tab.at[tab_rows(k)], uchunk, sem2),
                pltpu.async_copy(q_tab.at[tab_rows(k)], qchunk, sem3),
            ]
            for c in dls:
                c.wait()

            def qc2(i, _):
                qchunk2[i // 8, pl.ds((i % 8) * _L, _L)] = (
                    qchunk[pl.ds(i * _L, _L)])
                return 0

            lax.fori_loop(0, _NG, qc2, 0)
            _row_scale_inplace(sl, uchunk)
            plsc.subcore_barrier()
            copies = [
                pltpu.async_copy(sl.at[pl.ds(j * 128, 128)],
                                 s_sh.at[qchunk2.at[j]], sem, add=True)
                for j in range(_NIDX)
            ]
            for c in copies:
                c.wait()
            return 0

        lax.fori_loop(0, n_eff, up_body, 0)

        plsc.subcore_barrier()
        pltpu.sync_copy(s_sh.at[rows], sl)
        pltpu.sync_copy(m_sh.at[rows], mchunk)
        _row_scale_inplace(sl, mchunk)
        pltpu.sync_copy(sl, s_sh.at[rows])

        def down_body(k, _):
            plsc.subcore_barrier()
            dls = [
                pltpu.async_copy(u_tab.at[tab_rows(k)], uchunk, sem2),
                pltpu.async_copy(q_tab.at[tab_rows(k)], qchunk, sem3),
            ]
            for c in dls:
                c.wait()
            copies = [
                pltpu.async_copy(s_sh.at[qchunk.at[pl.ds(j * 128, 128)]],
                                 g.at[pl.ds(j * 128, 128)], sem)
                for j in range(_NIDX)
            ]
            for c in copies:
                c.wait()
            plsc.subcore_barrier()

            def fma_body(grp, _):
                r0 = grp * _L
                uv = uchunk[pl.ds(r0, _L)]
                for l in range(_L):
                    sc = jnp.broadcast_to(uv[l], (_L,))
                    for j in range(_CP // _L):
                        s_ = pl.ds(j * _L, _L)
                        sl[r0 + l, s_] = sl[r0 + l, s_] + sc * g[r0 + l, s_]
                return 0

            lax.fori_loop(0, _NG, fma_body, 0)
            pltpu.sync_copy(sl, s_sh.at[rows])
            return 0

        lax.fori_loop(0, n_eff, down_body, 0)

        def out_body(r, _):
            invv = 1.0 / sl[r, pl.ds(_C, _L)]
            inv = jnp.broadcast_to(invv[0], (_L,))
            for j in range(_CP // _L):
                s_ = pl.ds(j * _L, _L)
                sl[r, s_] = sl[r, s_] * inv
            return 0

        lax.fori_loop(0, _CHUNK, out_body, 0)
        pltpu.sync_copy(sl, out_hbm.at[hrows])
        return 0

    lax.fori_loop(0, _B // _NC, rep_body, 0)


_mesh = plsc.VectorSubcoreMesh(
    core_axis_name="c", subcore_axis_name="s", num_cores=_NC, num_subcores=_NS)

_filter_call = pl.kernel(
    _tree_filter_kernel,
    out_type=jax.ShapeDtypeStruct((_B * _V, _CP), jnp.float32),
    mesh=_mesh,
    compiler_params=pltpu.CompilerParams(
        needs_layout_passes=False, use_tc_tiling_on_sc=False),
    scratch_types=[
        pltpu.VMEM_SHARED((_V, _CP), jnp.float32),
        pltpu.HBM((_B * _R * _V,), jnp.float32),
        pltpu.HBM((_B * _R * _V,), jnp.int32),
        pltpu.VMEM_SHARED((_V,), jnp.float32),
        pltpu.VMEM_SHARED((_NS * _L,), jnp.float32),
        pltpu.VMEM((_CHUNK, _CP), jnp.float32),
        pltpu.VMEM((_CHUNK, _CP), jnp.float32),
        pltpu.VMEM((_CHUNK, _C), jnp.float32),
        pltpu.VMEM((_V,), jnp.float32),
        pltpu.VMEM((_V,), jnp.int32),
        pltpu.VMEM((_CHUNK,), jnp.float32),
        pltpu.VMEM((_CHUNK,), jnp.int32),
        pltpu.VMEM((_NIDX, 128), jnp.int32),
        pltpu.VMEM((_CHUNK,), jnp.float32),
        pltpu.VMEM((_L,), jnp.float32),
        pltpu.SemaphoreType.DMA,
        pltpu.SemaphoreType.DMA,
        pltpu.SemaphoreType.DMA,
    ],
)


@jax.jit
def kernel(feature_in, embed_in, tree):
    B, C, H, W = feature_in.shape
    V = H * W
    x = feature_in.reshape(B, C, V).transpose(0, 2, 1).reshape(B * V, C)
    e = embed_in.reshape(B, C, V).transpose(0, 2, 1)
    e = jnp.concatenate([e, jnp.zeros((B, V, _CP - C), jnp.float32)], axis=2)
    e = e.reshape(B * V, _CP)
    tree2 = tree.astype(jnp.int32).reshape(B * V // 128, 128)
    out = _filter_call(x, e, tree2)
    out = out.reshape(B, V, _CP)
    return out[:, :, :_C].transpose(0, 2, 1).reshape(B, C, H, W)


# speedup vs baseline: 319.7543x; 1.0132x over previous
import jax
import jax.numpy as jnp
from jax import lax
from jax.experimental import pallas as pl
from jax.experimental.pallas import tpu as pltpu
from jax.experimental.pallas import tpu_sc as plsc

_SIGMA = 0.02
_B = 4
_V = 4096
_C = 96
_CP = 112
_R = 12
_NC = 2
_NS = 16
_CHUNK = _V // _NS
_NIDX = _CHUNK // 128
_L = 16
_NG = _CHUNK // _L


def _row_scale_inplace(buf, scale_ref):
    def body(grp, _):
        r0 = grp * _L
        sv = scale_ref[pl.ds(r0, _L)]
        for l in range(_L):
            sc = jnp.broadcast_to(sv[l], (_L,))
            for j in range(_CP // _L):
                s_ = pl.ds(j * _L, _L)
                buf[r0 + l, s_] = buf[r0 + l, s_] * sc
        return 0

    lax.fori_loop(0, _NG, body, 0)


def _tree_filter_kernel(x_hbm, e_hbm, tree2_hbm, out_hbm,
                        s_sh, u_tab, q_tab, m_sh, flags_sh,
                        sl, g, xo, ufull, qfull,
                        uchunk, qchunk, qchunk2, mchunk, fbuf, sem, sem2, sem3):
    cid = lax.axis_index("c")
    tib = lax.axis_index("s")
    base = tib * _CHUNK
    rows = pl.ds(base, _CHUNK)
    iota = lax.iota(jnp.int32, _L)

    def rep_body(rep, _):
        b = cid + _NC * rep
        hrows = pl.ds(b * _V + base, _CHUNK)
        hoff = jnp.broadcast_to(b * _V, (_L,)).astype(jnp.int32)

        def tab_rows(k):
            return pl.ds((b * _R + k) * _V + base, _CHUNK)

        def tab_full(k):
            return pl.ds((b * _R + k) * _V, _V)

        plsc.subcore_barrier()

        dls0 = [
            pltpu.async_copy(x_hbm.at[hrows], xo, sem2),
            pltpu.async_copy(
                tree2_hbm.at[pl.ds((b * _V) // 128 + tib * _NIDX, _NIDX)],
                qchunk2, sem3),
        ]
        for c in dls0:
            c.wait()
        onesv = jnp.where(iota == 0, 1.0, 0.0).astype(jnp.float32)

        def xg_body(grp, _):
            r0 = grp * _L
            for l in range(_L):
                for j in range(_C // _L):
                    s_ = pl.ds(j * _L, _L)
                    sl[r0 + l, s_] = xo[r0 + l, s_]
                sl[r0 + l, pl.ds(_C, _L)] = onesv
            return 0

        lax.fori_loop(0, _NG, xg_body, 0)
        pltpu.sync_copy(sl, s_sh.at[rows])

        def qc_body(i, _):
            s_ = pl.ds((i % 8) * _L, _L)
            qchunk[pl.ds(i * _L, _L)] = qchunk2[i // 8, s_]
            qchunk2[i // 8, s_] = qchunk2[i // 8, s_] + hoff
            return 0

        lax.fori_loop(0, _NG, qc_body, 0)

        own = pltpu.async_copy(e_hbm.at[hrows], sl, sem2)
        copies = [
            pltpu.async_copy(e_hbm.at[qchunk2.at[j]],
                             g.at[pl.ds(j * 128, 128)], sem)
            for j in range(_NIDX)
        ]
        own.wait()
        for c in copies:
            c.wait()

        def wgrp_body(grp, _):
            r0 = grp * _L
            d2v = jnp.zeros((_L,), jnp.float32)
            for l in range(_L):
                acc = jnp.zeros((_L,), jnp.float32)
                for j in range(_CP // _L):
                    s_ = pl.ds(j * _L, _L)
                    d = sl[r0 + l, s_] - g[r0 + l, s_]
                    acc = acc + d * d
                for sh in (1, 2, 4, 8):
                    acc = acc + acc.at[iota ^ sh].get(
                        mode="promise_in_bounds")
                d2v = jnp.where(iota == l, acc, d2v)
            wv = jnp.exp(d2v * (-1.0 / _SIGMA))
            row0 = (iota == 0) & jnp.broadcast_to(base + r0 == 0, (_L,))
            uchunk[pl.ds(r0, _L)] = jnp.where(row0, 0.0, wv)
            mchunk[pl.ds(r0, _L)] = jnp.where(row0, 1.0, 1.0 - wv * wv)
            return 0

        lax.fori_loop(0, _NG, wgrp_body, 0)

        ups0 = [
            pltpu.async_copy(uchunk, u_tab.at[tab_rows(0)], sem),
            pltpu.async_copy(qchunk, q_tab.at[tab_rows(0)], sem2),
            pltpu.async_copy(mchunk, m_sh.at[rows], sem3),
        ]
        for c in ups0:
            c.wait()

        def build_cond(carry):
            return carry[1]

        def build_body(carry):
            k, _, nz = carry
            plsc.subcore_barrier()
            dls = [
                pltpu.async_copy(u_tab.at[tab_full(k - 1)], ufull, sem2),
                pltpu.async_copy(q_tab.at[tab_full(k - 1)], qfull, sem3),
            ]
            for c in dls:
                c.wait()

            def grp(i, maxv):
                s_ = pl.ds(i * _L, _L)
                qv = qfull[pl.ds(base + i * _L, _L)]
                uv = ufull[pl.ds(base + i * _L, _L)]
                uq = plsc.load_gather(ufull, [qv])
                qq = plsc.load_gather(qfull, [qv])
                u2 = uv * uq
                uchunk[s_] = u2
                qchunk[s_] = qq
                return jnp.maximum(maxv, u2)

            maxv = lax.fori_loop(0, _NG, grp,
                                 jnp.zeros((_L,), jnp.float32))
            fbuf[pl.ds(0, _L)] = maxv
            ups = [
                pltpu.async_copy(fbuf, flags_sh.at[pl.ds(tib * _L, _L)], sem),
                pltpu.async_copy(uchunk, u_tab.at[tab_rows(k)], sem2),
                pltpu.async_copy(qchunk, q_tab.at[tab_rows(k)], sem3),
            ]
            for c in ups:
                c.wait()
            plsc.subcore_barrier()
            pltpu.sync_copy(flags_sh, ufull.at[pl.ds(0, _NS * _L)])
            mk = ufull[pl.ds(0, _L)]
            for t in range(1, _NS):
                mk = jnp.maximum(mk, ufull[pl.ds(t * _L, _L)])
            for sh in (1, 2, 4, 8):
                mk = jnp.maximum(
                    mk, mk.at[iota ^ sh].get(mode="promise_in_bounds"))
            nonzero = mk[0] > 0.0
            cont = jnp.logical_and(nonzero, k + 1 < _R)
            return (k + 1, cont, nz + jnp.where(nonzero, 1, 0))

        _, _, n_eff = lax.while_loop(
            build_cond, build_body,
            (jnp.int32(1), jnp.bool_(True), jnp.int32(1)))

        def up_body(kk, _):
            k = n_eff - 1 - kk
            plsc.subcore_barrier()
            dls = [
                pltpu.async_copy(s_sh.at[rows], sl, sem),
                pltpu.async_copy(u_tab.at[tab_rows(k)], uchunk, sem2),
                pltpu.async_copy(q_tab.at[tab_rows(k)], qchunk, sem3),
            ]
            for c in dls:
                c.wait()

            def qc2(i, _):
                qchunk2[i // 8, pl.ds((i % 8) * _L, _L)] = (
                    qchunk[pl.ds(i * _L, _L)])
                return 0

            lax.fori_loop(0, _NG, qc2, 0)
            _row_scale_inplace(sl, uchunk)
            plsc.subcore_barrier()
            copies = [
                pltpu.async_copy(sl.at[pl.ds(j * 128, 128)],
                                 s_sh.at[qchunk2.at[j]], sem, add=True)
                for j in range(_NIDX)
            ]
            for c in copies:
                c.wait()
            return 0

        lax.fori_loop(0, n_eff, up_body, 0)

        plsc.subcore_barrier()
        dlm = [
            pltpu.async_copy(s_sh.at[rows], sl, sem),
            pltpu.async_copy(m_sh.at[rows], mchunk, sem2),
        ]
        for c in dlm:
            c.wait()
        _row_scale_inplace(sl, mchunk)
        pltpu.sync_copy(sl, s_sh.at[rows])

        def down_body(k, _):
            plsc.subcore_barrier()
            dls = [
                pltpu.async_copy(u_tab.at[tab_rows(k)], uchunk, sem2),
                pltpu.async_copy(q_tab.at[tab_rows(k)], qchunk, sem3),
            ]
            for c in dls:
                c.wait()
            copies = [
                pltpu.async_copy(s_sh.at[qchunk.at[pl.ds(j * 128, 128)]],
                                 g.at[pl.ds(j * 128, 128)], sem)
                for j in range(_NIDX)
            ]
            for c in copies:
                c.wait()
            plsc.subcore_barrier()

            def fma_body(grp, _):
                r0 = grp * _L
                uv = uchunk[pl.ds(r0, _L)]
                for l in range(_L):
                    sc = jnp.broadcast_to(uv[l], (_L,))
                    for j in range(_CP // _L):
                        s_ = pl.ds(j * _L, _L)
                        sl[r0 + l, s_] = sl[r0 + l, s_] + sc * g[r0 + l, s_]
                return 0

            lax.fori_loop(0, _NG, fma_body, 0)
            pltpu.sync_copy(sl, s_sh.at[rows])
            return 0

        lax.fori_loop(0, n_eff, down_body, 0)

        def out_body(r, _):
            invv = 1.0 / sl[r, pl.ds(_C, _L)]
            inv = jnp.broadcast_to(invv[0], (_L,))
            for j in range(_CP // _L):
                s_ = pl.ds(j * _L, _L)
                sl[r, s_] = sl[r, s_] * inv
            return 0

        lax.fori_loop(0, _CHUNK, out_body, 0)
        pltpu.sync_copy(sl, out_hbm.at[hrows])
        return 0

    lax.fori_loop(0, _B // _NC, rep_body, 0)


_mesh = plsc.VectorSubcoreMesh(
    core_axis_name="c", subcore_axis_name="s", num_cores=_NC, num_subcores=_NS)

_filter_call = pl.kernel(
    _tree_filter_kernel,
    out_type=jax.ShapeDtypeStruct((_B * _V, _CP), jnp.float32),
    mesh=_mesh,
    compiler_params=pltpu.CompilerParams(
        needs_layout_passes=False, use_tc_tiling_on_sc=False),
    scratch_types=[
        pltpu.VMEM_SHARED((_V, _CP), jnp.float32),
        pltpu.HBM((_B * _R * _V,), jnp.float32),
        pltpu.HBM((_B * _R * _V,), jnp.int32),
        pltpu.VMEM_SHARED((_V,), jnp.float32),
        pltpu.VMEM_SHARED((_NS * _L,), jnp.float32),
        pltpu.VMEM((_CHUNK, _CP), jnp.float32),
        pltpu.VMEM((_CHUNK, _CP), jnp.float32),
        pltpu.VMEM((_CHUNK, _C), jnp.float32),
        pltpu.VMEM((_V,), jnp.float32),
        pltpu.VMEM((_V,), jnp.int32),
        pltpu.VMEM((_CHUNK,), jnp.float32),
        pltpu.VMEM((_CHUNK,), jnp.int32),
        pltpu.VMEM((_NIDX, 128), jnp.int32),
        pltpu.VMEM((_CHUNK,), jnp.float32),
        pltpu.VMEM((_L,), jnp.float32),
        pltpu.SemaphoreType.DMA,
        pltpu.SemaphoreType.DMA,
        pltpu.SemaphoreType.DMA,
    ],
)


@jax.jit
def kernel(feature_in, embed_in, tree):
    B, C, H, W = feature_in.shape
    V = H * W
    x = feature_in.reshape(B, C, V).transpose(0, 2, 1).reshape(B * V, C)
    e = embed_in.reshape(B, C, V).transpose(0, 2, 1)
    e = jnp.concatenate([e, jnp.zeros((B, V, _CP - C), jnp.float32)], axis=2)
    e = e.reshape(B * V, _CP)
    tree2 = tree.astype(jnp.int32).reshape(B * V // 128, 128)
    out = _filter_call(x, e, tree2)
    out = out.reshape(B, V, _CP)
    return out[:, :, :_C].transpose(0, 2, 1).reshape(B, C, H, W)


# speedup vs baseline: 320.9644x; 1.0038x over previous
import jax
import jax.numpy as jnp
from jax import lax
from jax.experimental import pallas as pl
from jax.experimental.pallas import tpu as pltpu
from jax.experimental.pallas import tpu_sc as plsc

_SIGMA = 0.02
_B = 4
_V = 4096
_C = 96
_CP = 112
_R = 12
_NC = 2
_NS = 16
_CHUNK = _V // _NS
_NIDX = _CHUNK // 128
_L = 16
_NG = _CHUNK // _L


def _row_scale_inplace(buf, scale_ref):
    def body(grp, _):
        r0 = grp * _L
        sv = scale_ref[pl.ds(r0, _L)]
        for l in range(_L):
            sc = jnp.broadcast_to(sv[l], (_L,))
            for j in range(_CP // _L):
                s_ = pl.ds(j * _L, _L)
                buf[r0 + l, s_] = buf[r0 + l, s_] * sc
        return 0

    lax.fori_loop(0, _NG, body, 0)


def _tree_filter_kernel(x_hbm, e_hbm, tree2_hbm, out_hbm,
                        s_sh, u_tab, q_tab, m_sh, flags_sh,
                        sl, g, xo, ufull, qfull,
                        uchunk, qchunk, qchunk2, mchunk, fbuf, sem, sem2, sem3):
    cid = lax.axis_index("c")
    tib = lax.axis_index("s")
    base = tib * _CHUNK
    rows = pl.ds(base, _CHUNK)
    iota = lax.iota(jnp.int32, _L)

    def rep_body(rep, _):
        b = cid + _NC * rep
        hrows = pl.ds(b * _V + base, _CHUNK)
        hoff = jnp.broadcast_to(b * _V, (_L,)).astype(jnp.int32)

        def tab_rows(k):
            return pl.ds((b * _R + k) * _V + base, _CHUNK)

        def tab_full(k):
            return pl.ds((b * _R + k) * _V, _V)

        plsc.subcore_barrier()

        dls0 = [
            pltpu.async_copy(x_hbm.at[hrows], xo, sem2),
            pltpu.async_copy(
                tree2_hbm.at[pl.ds((b * _V) // 128 + tib * _NIDX, _NIDX)],
                qchunk2, sem3),
        ]
        for c in dls0:
            c.wait()
        onesv = jnp.where(iota == 0, 1.0, 0.0).astype(jnp.float32)

        def xg_body(grp, _):
            r0 = grp * _L
            for l in range(_L):
                for j in range(_C // _L):
                    s_ = pl.ds(j * _L, _L)
                    sl[r0 + l, s_] = xo[r0 + l, s_]
                sl[r0 + l, pl.ds(_C, _L)] = onesv
            return 0

        lax.fori_loop(0, _NG, xg_body, 0)
        pltpu.sync_copy(sl, s_sh.at[rows])

        def qc_body(i, _):
            s_ = pl.ds((i % 8) * _L, _L)
            qchunk[pl.ds(i * _L, _L)] = qchunk2[i // 8, s_]
            qchunk2[i // 8, s_] = qchunk2[i // 8, s_] + hoff
            return 0

        lax.fori_loop(0, _NG, qc_body, 0)

        own = pltpu.async_copy(e_hbm.at[hrows], sl, sem2)
        copies = [
            pltpu.async_copy(e_hbm.at[qchunk2.at[j]],
                             g.at[pl.ds(j * 128, 128)], sem)
            for j in range(_NIDX)
        ]
        own.wait()
        for c in copies:
            c.wait()

        def wgrp_body(grp, _):
            r0 = grp * _L
            d2v = jnp.zeros((_L,), jnp.float32)
            for l in range(_L):
                acc = jnp.zeros((_L,), jnp.float32)
                for j in range(_CP // _L):
                    s_ = pl.ds(j * _L, _L)
                    d = sl[r0 + l, s_] - g[r0 + l, s_]
                    acc = acc + d * d
                for sh in (1, 2, 4, 8):
                    acc = acc + acc.at[iota ^ sh].get(
                        mode="promise_in_bounds")
                d2v = jnp.where(iota == l, acc, d2v)
            wv = jnp.exp(d2v * (-1.0 / _SIGMA))
            row0 = (iota == 0) & jnp.broadcast_to(base + r0 == 0, (_L,))
            uchunk[pl.ds(r0, _L)] = jnp.where(row0, 0.0, wv)
            mchunk[pl.ds(r0, _L)] = jnp.where(row0, 1.0, 1.0 - wv * wv)
            return 0

        lax.fori_loop(0, _NG, wgrp_body, 0)

        ups0 = [
            pltpu.async_copy(uchunk, u_tab.at[tab_rows(0)], sem),
            pltpu.async_copy(qchunk, q_tab.at[tab_rows(0)], sem2),
            pltpu.async_copy(mchunk, m_sh.at[rows], sem3),
        ]
        for c in ups0:
            c.wait()

        plsc.subcore_barrier()

        def build_cond(carry):
            return carry[1]

        def build_body(carry):
            k, _, nz = carry
            dls = [
                pltpu.async_copy(u_tab.at[tab_full(k - 1)], ufull, sem2),
                pltpu.async_copy(q_tab.at[tab_full(k - 1)], qfull, sem3),
            ]
            for c in dls:
                c.wait()

            def grp(i, maxv):
                s_ = pl.ds(i * _L, _L)
                qv = qfull[pl.ds(base + i * _L, _L)]
                uv = ufull[pl.ds(base + i * _L, _L)]
                uq = plsc.load_gather(ufull, [qv])
                qq = plsc.load_gather(qfull, [qv])
                u2 = uv * uq
                uchunk[s_] = u2
                qchunk[s_] = qq
                return jnp.maximum(maxv, u2)

            maxv = lax.fori_loop(0, _NG, grp,
                                 jnp.zeros((_L,), jnp.float32))
            fbuf[pl.ds(0, _L)] = maxv
            ups = [
                pltpu.async_copy(fbuf, flags_sh.at[pl.ds(tib * _L, _L)], sem),
                pltpu.async_copy(uchunk, u_tab.at[tab_rows(k)], sem2),
                pltpu.async_copy(qchunk, q_tab.at[tab_rows(k)], sem3),
            ]
            for c in ups:
                c.wait()
            plsc.subcore_barrier()
            pltpu.sync_copy(flags_sh, ufull.at[pl.ds(0, _NS * _L)])
            mk = ufull[pl.ds(0, _L)]
            for t in range(1, _NS):
                mk = jnp.maximum(mk, ufull[pl.ds(t * _L, _L)])
            for sh in (1, 2, 4, 8):
                mk = jnp.maximum(
                    mk, mk.at[iota ^ sh].get(mode="promise_in_bounds"))
            nonzero = mk[0] > 0.0
            cont = jnp.logical_and(nonzero, k + 1 < _R)
            return (k + 1, cont, nz + jnp.where(nonzero, 1, 0))

        _, _, n_eff = lax.while_loop(
            build_cond, build_body,
            (jnp.int32(1), jnp.bool_(True), jnp.int32(1)))

        def up_body(kk, _):
            k = n_eff - 1 - kk
            plsc.subcore_barrier()
            dls = [
                pltpu.async_copy(s_sh.at[rows], sl, sem),
                pltpu.async_copy(u_tab.at[tab_rows(k)], uchunk, sem2),
                pltpu.async_copy(q_tab.at[tab_rows(k)], qchunk, sem3),
            ]
            for c in dls:
                c.wait()

            def qc2(i, _):
                qchunk2[i // 8, pl.ds((i % 8) * _L, _L)] = (
                    qchunk[pl.ds(i * _L, _L)])
                return 0

            lax.fori_loop(0, _NG, qc2, 0)
            _row_scale_inplace(sl, uchunk)
            plsc.subcore_barrier()
            copies = [
                pltpu.async_copy(sl.at[pl.ds(j * 128, 128)],
                                 s_sh.at[qchunk2.at[j]], sem, add=True)
                for j in range(_NIDX)
            ]
            for c in copies:
                c.wait()
            return 0

        lax.fori_loop(0, n_eff, up_body, 0)

        plsc.subcore_barrier()
        dlm = [
            pltpu.async_copy(s_sh.at[rows], sl, sem),
            pltpu.async_copy(m_sh.at[rows], mchunk, sem2),
        ]
        for c in dlm:
            c.wait()
        _row_scale_inplace(sl, mchunk)
        pltpu.sync_copy(sl, s_sh.at[rows])

        def down_body(k, _):
            plsc.subcore_barrier()
            dls = [
                pltpu.async_copy(u_tab.at[tab_rows(k)], uchunk, sem2),
                pltpu.async_copy(q_tab.at[tab_rows(k)], qchunk, sem3),
            ]
            for c in dls:
                c.wait()
            copies = [
                pltpu.async_copy(s_sh.at[qchunk.at[pl.ds(j * 128, 128)]],
                                 g.at[pl.ds(j * 128, 128)], sem)
                for j in range(_NIDX)
            ]
            for c in copies:
                c.wait()
            plsc.subcore_barrier()

            def fma_body(grp, _):
                r0 = grp * _L
                uv = uchunk[pl.ds(r0, _L)]
                for l in range(_L):
                    sc = jnp.broadcast_to(uv[l], (_L,))
                    for j in range(_CP // _L):
                        s_ = pl.ds(j * _L, _L)
                        sl[r0 + l, s_] = sl[r0 + l, s_] + sc * g[r0 + l, s_]
                return 0

            lax.fori_loop(0, _NG, fma_body, 0)
            pltpu.sync_copy(sl, s_sh.at[rows])
            return 0

        lax.fori_loop(0, n_eff, down_body, 0)

        def out_body(r, _):
            invv = 1.0 / sl[r, pl.ds(_C, _L)]
            inv = jnp.broadcast_to(invv[0], (_L,))
            for j in range(_CP // _L):
                s_ = pl.ds(j * _L, _L)
                sl[r, s_] = sl[r, s_] * inv
            return 0

        lax.fori_loop(0, _CHUNK, out_body, 0)
        pltpu.sync_copy(sl, out_hbm.at[hrows])
        return 0

    lax.fori_loop(0, _B // _NC, rep_body, 0)


_mesh = plsc.VectorSubcoreMesh(
    core_axis_name="c", subcore_axis_name="s", num_cores=_NC, num_subcores=_NS)

_filter_call = pl.kernel(
    _tree_filter_kernel,
    out_type=jax.ShapeDtypeStruct((_B * _V, _CP), jnp.float32),
    mesh=_mesh,
    compiler_params=pltpu.CompilerParams(
        needs_layout_passes=False, use_tc_tiling_on_sc=False),
    scratch_types=[
        pltpu.VMEM_SHARED((_V, _CP), jnp.float32),
        pltpu.HBM((_B * _R * _V,), jnp.float32),
        pltpu.HBM((_B * _R * _V,), jnp.int32),
        pltpu.VMEM_SHARED((_V,), jnp.float32),
        pltpu.VMEM_SHARED((_NS * _L,), jnp.float32),
        pltpu.VMEM((_CHUNK, _CP), jnp.float32),
        pltpu.VMEM((_CHUNK, _CP), jnp.float32),
        pltpu.VMEM((_CHUNK, _C), jnp.float32),
        pltpu.VMEM((_V,), jnp.float32),
        pltpu.VMEM((_V,), jnp.int32),
        pltpu.VMEM((_CHUNK,), jnp.float32),
        pltpu.VMEM((_CHUNK,), jnp.int32),
        pltpu.VMEM((_NIDX, 128), jnp.int32),
        pltpu.VMEM((_CHUNK,), jnp.float32),
        pltpu.VMEM((_L,), jnp.float32),
        pltpu.SemaphoreType.DMA,
        pltpu.SemaphoreType.DMA,
        pltpu.SemaphoreType.DMA,
    ],
)


@jax.jit
def kernel(feature_in, embed_in, tree):
    B, C, H, W = feature_in.shape
    V = H * W
    x = feature_in.reshape(B, C, V).transpose(0, 2, 1).reshape(B * V, C)
    e = embed_in.reshape(B, C, V).transpose(0, 2, 1)
    e = jnp.concatenate([e, jnp.zeros((B, V, _CP - C), jnp.float32)], axis=2)
    e = e.reshape(B * V, _CP)
    tree2 = tree.astype(jnp.int32).reshape(B * V // 128, 128)
    out = _filter_call(x, e, tree2)
    out = out.reshape(B, V, _CP)
    return out[:, :, :_C].transpose(0, 2, 1).reshape(B, C, H, W)
